# baseline (speedup 1.0000x reference)
import jax
import jax.numpy as jnp
from jax import lax
from jax.experimental import pallas as pl
from jax.experimental.pallas import tpu as pltpu
from jax.experimental.pallas import tpu_sc as plsc

BATCH = 1024
HIDDEN = 32
NUM_ENT = 100000

_info = plsc.get_sparse_core_info()
_NC, _NS = _info.num_cores, _info.num_subcores
_NW = _NC * _NS
_BPW = BATCH // _NW


def _sc_ids(ent_hbm, rel_hbm, ewm_hbm, rel_emb_hbm,
            w1o_hbm, w2o_hbm, m3_hbm,
            ent_v, relid_v, w1i_v, w2i_v, relr_v, sem1, sem2, sem3):
    wid = lax.axis_index("s") * _NC + lax.axis_index("c")
    base = wid * _BPW

    pltpu.sync_copy(ent_hbm.at[pl.ds(base, _BPW)], ent_v)
    pltpu.sync_copy(rel_hbm.at[pl.ds(base, _BPW)], relid_v)
    cp_rel = pltpu.async_copy(rel_emb_hbm.at[relid_v], relr_v, sem2)

    for j in range(_BPW // 16):
        e = ent_v[pl.ds(16 * j, 16)]
        w1i_v[pl.ds(16 * j, 16)] = e
        w2i_v[pl.ds(16 * j, 16)] = e + NUM_ENT
    cp1 = pltpu.async_copy(ewm_hbm.at[w1i_v], ent_v, sem1)
    cp2 = pltpu.async_copy(ewm_hbm.at[w2i_v], w1i_v, sem3)
    cp1.wait()
    pltpu.sync_copy(ent_v, w1o_hbm.at[pl.ds(base, _BPW)])
    cp2.wait()
    pltpu.sync_copy(w1i_v, w2o_hbm.at[pl.ds(base, _BPW)])
    cp_rel.wait()
    pltpu.sync_copy(relr_v, m3_hbm.at[pl.ds(base, _BPW)])


def _sc_lookup_ids(ent_ids, rel_ids, ewm_cm, rel_embed):
    mesh = plsc.VectorSubcoreMesh(core_axis_name="c", subcore_axis_name="s")
    run = pl.kernel(
        _sc_ids,
        mesh=mesh,
        compiler_params=pltpu.CompilerParams(use_tc_tiling_on_sc=False),
        out_type=(
            jax.ShapeDtypeStruct((BATCH,), jnp.int32),
            jax.ShapeDtypeStruct((BATCH,), jnp.int32),
            jax.ShapeDtypeStruct((BATCH, HIDDEN), jnp.float32),
        ),
        scratch_types=[
            pltpu.VMEM((_BPW,), jnp.int32),
            pltpu.VMEM((_BPW,), jnp.int32),
            pltpu.VMEM((_BPW,), jnp.int32),
            pltpu.VMEM((_BPW,), jnp.int32),
            pltpu.VMEM((_BPW, HIDDEN), jnp.float32),
            pltpu.SemaphoreType.DMA,
            pltpu.SemaphoreType.DMA,
            pltpu.SemaphoreType.DMA,
        ],
    )
    return run(ent_ids, rel_ids, ewm_cm, rel_embed)


_NBUF = 32


def _wg_body(ids_ref, words_ref, o_ref, *rest):
    bufs = rest[:_NBUF]
    sems = rest[_NBUF:]
    n_ids = 2 * BATCH
    lanes = lax.broadcasted_iota(jnp.int32, (HIDDEN, 128), 1)

    def issue(k, slot):
        wid = ids_ref[k]
        tile = pl.multiple_of((wid // 128) * 128, 128)
        pltpu.make_async_copy(
            words_ref.at[:, pl.ds(tile, 128)], bufs[slot], sems[slot]
        ).start()

    for s in range(_NBUF):
        issue(s, s)

    for g in range(n_ids // 128):
        half = g // 8
        colb = g % 8

        def step(q, acc):
            a = acc
            for t in range(_NBUF):
                k = 128 * g + _NBUF * q + t
                pltpu.make_async_copy(
                    words_ref.at[:, pl.ds(0, 128)], bufs[t], sems[t]
                ).wait()
                col = ids_ref[k] % 128
                tgt = _NBUF * q + t
                rolled = pltpu.roll(bufs[t][...], tgt - col, 1)
                a = a + jnp.where(lanes == tgt, rolled, 0.0)

                @pl.when(k + _NBUF < n_ids)
                def _():
                    issue(k + _NBUF, t)
            return a

        acc = lax.fori_loop(0, 128 // _NBUF, step,
                            jnp.zeros((HIDDEN, 128), jnp.float32))
        o_ref[32 * half:32 * half + 32, 128 * colb:128 * colb + 128] = acc


def _tc_word_gather(wids, words_t):
    return pl.pallas_call(
        _wg_body,
        in_specs=[
            pl.BlockSpec(memory_space=pltpu.SMEM),
            pl.BlockSpec(memory_space=pl.ANY),
        ],
        out_specs=pl.BlockSpec(memory_space=pltpu.VMEM),
        out_shape=jax.ShapeDtypeStruct((2 * HIDDEN, BATCH), jnp.float32),
        scratch_shapes=(
            [pltpu.VMEM((HIDDEN, 128), jnp.float32)] * _NBUF
            + [pltpu.SemaphoreType.DMA] * _NBUF
        ),
    )(wids, words_t)


_N_BLK = 2048


def _mm_body(m12_ref, m3_ref, wt_ref, b_ref, o_ref):
    m12 = m12_ref[...].astype(jnp.bfloat16)
    w12 = wt_ref[0:2 * HIDDEN, :].astype(jnp.bfloat16)
    ones = jnp.ones((BATCH, 1), jnp.bfloat16)
    m3_aug = jnp.concatenate(
        [m3_ref[...].astype(jnp.bfloat16), ones], axis=1)
    w3_aug = jnp.concatenate(
        [wt_ref[2 * HIDDEN:, :], b_ref[...]], axis=0).astype(jnp.bfloat16)
    acc = lax.dot_general(w12, m12, (((0,), (0,)), ((), ())),
                          preferred_element_type=jnp.float32)
    acc += lax.dot_general(w3_aug, m3_aug, (((0,), (1,)), ((), ())),
                           preferred_element_type=jnp.float32)
    o_ref[...] = acc


def _project(m12t, m3, W, b):
    n = W.shape[0]
    w_t = W.T
    b2d = b.reshape(1, n)
    grid = (pl.cdiv(n, _N_BLK),)
    out_t = pl.pallas_call(
        _mm_body,
        grid=grid,
        in_specs=[
            pl.BlockSpec((2 * HIDDEN, BATCH), lambda i: (0, 0)),
            pl.BlockSpec((BATCH, HIDDEN), lambda i: (0, 0)),
            pl.BlockSpec((3 * HIDDEN, _N_BLK), lambda i: (0, i)),
            pl.BlockSpec((1, _N_BLK), lambda i: (0, i)),
        ],
        out_specs=pl.BlockSpec((_N_BLK, BATCH), lambda i: (i, 0)),
        out_shape=jax.ShapeDtypeStruct((n, BATCH), jnp.float32),
    )(m12t, m3, w_t, b2d)
    return out_t.T


@jax.jit
def kernel(batch_data, ent_word_map, words_embd, rel_embed, W, b):
    ent_ids = batch_data[:, 0]
    rel_ids = batch_data[:, 1]
    ewm_cm = ent_word_map.T.reshape(-1)
    w1, w2, m3 = _sc_lookup_ids(ent_ids, rel_ids, ewm_cm, rel_embed)
    wids = jnp.concatenate([w1, w2])
    m12t = _tc_word_gather(wids, words_embd.T)
    return _project(m12t, m3, W, b)


# speedup vs baseline: 1.0186x; 1.0186x over previous
import jax
import jax.numpy as jnp
from jax import lax
from jax.experimental import pallas as pl
from jax.experimental.pallas import tpu as pltpu
from jax.experimental.pallas import tpu_sc as plsc

BATCH = 1024
HIDDEN = 32
NUM_ENT = 100000

_info = plsc.get_sparse_core_info()
_NC, _NS = _info.num_cores, _info.num_subcores
_NW = _NC * _NS
_BPW = BATCH // _NW


def _sc_ids(ent_hbm, rel_hbm, ewm_hbm, rel_emb_hbm,
            w1o_hbm, w2o_hbm, m3_hbm,
            ent_v, relid_v, w1i_v, w2i_v, relr_v, sem1, sem2, sem3):
    wid = lax.axis_index("s") * _NC + lax.axis_index("c")
    base = wid * _BPW

    pltpu.sync_copy(ent_hbm.at[pl.ds(base, _BPW)], ent_v)
    pltpu.sync_copy(rel_hbm.at[pl.ds(base, _BPW)], relid_v)
    cp_rel = pltpu.async_copy(rel_emb_hbm.at[relid_v], relr_v, sem2)

    for j in range(_BPW // 16):
        e = ent_v[pl.ds(16 * j, 16)]
        w1i_v[pl.ds(16 * j, 16)] = e
        w2i_v[pl.ds(16 * j, 16)] = e + NUM_ENT
    cp1 = pltpu.async_copy(ewm_hbm.at[w1i_v], ent_v, sem1)
    cp2 = pltpu.async_copy(ewm_hbm.at[w2i_v], w1i_v, sem3)
    cp1.wait()
    pltpu.sync_copy(ent_v, w1o_hbm.at[pl.ds(base, _BPW)])
    cp2.wait()
    pltpu.sync_copy(w1i_v, w2o_hbm.at[pl.ds(base, _BPW)])
    cp_rel.wait()
    pltpu.sync_copy(relr_v, m3_hbm.at[pl.ds(base, _BPW)])


def _sc_lookup_ids(ent_ids, rel_ids, ewm_cm, rel_embed):
    mesh = plsc.VectorSubcoreMesh(core_axis_name="c", subcore_axis_name="s")
    run = pl.kernel(
        _sc_ids,
        mesh=mesh,
        compiler_params=pltpu.CompilerParams(use_tc_tiling_on_sc=False),
        out_type=(
            jax.ShapeDtypeStruct((BATCH,), jnp.int32),
            jax.ShapeDtypeStruct((BATCH,), jnp.int32),
            jax.ShapeDtypeStruct((BATCH, HIDDEN), jnp.float32),
        ),
        scratch_types=[
            pltpu.VMEM((_BPW,), jnp.int32),
            pltpu.VMEM((_BPW,), jnp.int32),
            pltpu.VMEM((_BPW,), jnp.int32),
            pltpu.VMEM((_BPW,), jnp.int32),
            pltpu.VMEM((_BPW, HIDDEN), jnp.float32),
            pltpu.SemaphoreType.DMA,
            pltpu.SemaphoreType.DMA,
            pltpu.SemaphoreType.DMA,
        ],
    )
    return run(ent_ids, rel_ids, ewm_cm, rel_embed)


_NBUF = 32


def _wg_body(ids_ref, words_ref, o_ref, *rest):
    bufs = rest[:_NBUF]
    sems = rest[_NBUF:]
    n_ids = 2 * BATCH
    lanes = lax.broadcasted_iota(jnp.int32, (HIDDEN, 128), 1)

    def issue(k, slot):
        wid = ids_ref[k]
        tile = pl.multiple_of((wid // 128) * 128, 128)
        pltpu.make_async_copy(
            words_ref.at[:, pl.ds(tile, 128)], bufs[slot], sems[slot]
        ).start()

    for s in range(_NBUF):
        issue(s, s)

    for g in range(n_ids // 128):
        half = g // 8
        colb = g % 8

        def step(q, acc):
            a = acc
            for t in range(_NBUF):
                k = 128 * g + _NBUF * q + t
                pltpu.make_async_copy(
                    words_ref.at[:, pl.ds(0, 128)], bufs[t], sems[t]
                ).wait()
                col = ids_ref[k] % 128
                tgt = _NBUF * q + t
                rolled = pltpu.roll(bufs[t][...], tgt - col, 1)
                a = a + jnp.where(lanes == tgt, rolled, 0.0)

                @pl.when(k + _NBUF < n_ids)
                def _():
                    issue(k + _NBUF, t)
            return a

        acc = lax.fori_loop(0, 128 // _NBUF, step,
                            jnp.zeros((HIDDEN, 128), jnp.float32))
        o_ref[32 * half:32 * half + 32, 128 * colb:128 * colb + 128] = acc


def _tc_word_gather(wids, words_t):
    return pl.pallas_call(
        _wg_body,
        in_specs=[
            pl.BlockSpec(memory_space=pltpu.SMEM),
            pl.BlockSpec(memory_space=pl.ANY),
        ],
        out_specs=pl.BlockSpec(memory_space=pltpu.VMEM),
        out_shape=jax.ShapeDtypeStruct((2 * HIDDEN, BATCH), jnp.float32),
        scratch_shapes=(
            [pltpu.VMEM((HIDDEN, 128), jnp.float32)] * _NBUF
            + [pltpu.SemaphoreType.DMA] * _NBUF
        ),
    )(wids, words_t)


_N_BLK = 4096


def _mm_body(m12_ref, m3_ref, wt_ref, b_ref, o_ref):
    m12 = m12_ref[...].astype(jnp.bfloat16)
    w12 = wt_ref[0:2 * HIDDEN, :].astype(jnp.bfloat16)
    ones = jnp.ones((BATCH, 1), jnp.bfloat16)
    m3_aug = jnp.concatenate(
        [m3_ref[...].astype(jnp.bfloat16), ones], axis=1)
    w3_aug = jnp.concatenate(
        [wt_ref[2 * HIDDEN:, :], b_ref[...]], axis=0).astype(jnp.bfloat16)
    acc = lax.dot_general(w12, m12, (((0,), (0,)), ((), ())),
                          preferred_element_type=jnp.float32)
    acc += lax.dot_general(w3_aug, m3_aug, (((0,), (1,)), ((), ())),
                           preferred_element_type=jnp.float32)
    o_ref[...] = acc


def _project(m12t, m3, W, b):
    n = W.shape[0]
    w_t = W.T
    b2d = b.reshape(1, n)
    grid = (pl.cdiv(n, _N_BLK),)
    out_t = pl.pallas_call(
        _mm_body,
        grid=grid,
        in_specs=[
            pl.BlockSpec((2 * HIDDEN, BATCH), lambda i: (0, 0)),
            pl.BlockSpec((BATCH, HIDDEN), lambda i: (0, 0)),
            pl.BlockSpec((3 * HIDDEN, _N_BLK), lambda i: (0, i)),
            pl.BlockSpec((1, _N_BLK), lambda i: (0, i)),
        ],
        out_specs=pl.BlockSpec((_N_BLK, BATCH), lambda i: (i, 0)),
        out_shape=jax.ShapeDtypeStruct((n, BATCH), jnp.float32),
    )(m12t, m3, w_t, b2d)
    return out_t.T


@jax.jit
def kernel(batch_data, ent_word_map, words_embd, rel_embed, W, b):
    ent_ids = batch_data[:, 0]
    rel_ids = batch_data[:, 1]
    ewm_cm = ent_word_map.T.reshape(-1)
    w1, w2, m3 = _sc_lookup_ids(ent_ids, rel_ids, ewm_cm, rel_embed)
    wids = jnp.concatenate([w1, w2])
    m12t = _tc_word_gather(wids, words_embd.T)
    return _project(m12t, m3, W, b)
